# baseline (speedup 1.0000x reference)
import functools
import math

import jax
import jax.numpy as jnp
from jax import lax
from jax.experimental import pallas as pl
from jax.experimental.pallas import tpu as pltpu
from jax.experimental.pallas import tpu_sc as plsc

EPS = 1e-5
B = 256


def _routing_tables(t32, N, E, NBLK):
    oh = (t32[:, None] == jnp.arange(E, dtype=jnp.int32)[None, :]).astype(jnp.int32)
    csum = jnp.cumsum(oh, axis=0)
    counts = csum[-1]
    within = jnp.sum(oh * csum, axis=1) - 1
    padded = ((counts + B - 1) // B) * B
    po = jnp.concatenate([jnp.zeros((1,), jnp.int32),
                          jnp.cumsum(padded)[:-1].astype(jnp.int32)])
    dest = jnp.sum(oh * po[None, :], axis=1) + within
    blk_e = jnp.sum((jnp.arange(NBLK, dtype=jnp.int32)[None, :]
                     >= (po // B)[1:, None]).astype(jnp.int32), axis=0)
    return dest, blk_e


def _sc_scatter_rows(x, dest, NPAD):
    N, D = x.shape
    info = plsc.get_sparse_core_info()
    NC, NS = info.num_cores, info.num_subcores
    NW = NC * NS
    TPW = N // NW
    CH = 32
    NCH = TPW // CH
    mesh = plsc.VectorSubcoreMesh(core_axis_name="c", subcore_axis_name="s")
    dest3 = dest.reshape(NW, NCH, CH)

    @functools.partial(
        pl.kernel, mesh=mesh,
        out_type=jax.ShapeDtypeStruct((NPAD, D), x.dtype),
        scratch_types=[
            pltpu.VMEM((NCH, CH), jnp.int32),
            pltpu.VMEM((3, CH, D), x.dtype),
            pltpu.SemaphoreType.DMA,
            pltpu.SemaphoreType.DMA,
            pltpu.SemaphoreType.DMA,
            pltpu.SemaphoreType.DMA,
        ],
    )
    def scatter_rows(x_hbm, dest_hbm, out_hbm, d_v, rows_v,
                     sem_i0, sem_i1, sem_i2, sem_o):
        wid = lax.axis_index("s") * NC + lax.axis_index("c")
        base = wid * TPW
        pltpu.sync_copy(dest_hbm.at[wid], d_v)
        sems = (sem_i0, sem_i1, sem_i2)

        def start_in(i, b):
            return pltpu.async_copy(
                x_hbm.at[pl.ds(base + i * CH, CH)], rows_v.at[b], sems[b])

        def start_out(i, b):
            return pltpu.async_copy(
                rows_v.at[b], out_hbm.at[d_v.at[i]], sem_o)

        NB = 3
        h_in = {0: start_in(0, 0), 1: start_in(1, 1)}
        h_out = {}
        for i in range(NCH):
            b = i % NB
            h_in.pop(i).wait()
            if i + 2 < NCH:
                nb = (i + 2) % NB
                if h_out.get(nb) is not None:
                    h_out.pop(nb).wait()
                h_in[i + 2] = start_in(i + 2, nb)
            h_out[b] = start_out(i, b)
        for b, h in list(h_out.items()):
            h.wait()

    return scatter_rows(x, dest3)


def _sc_unpermute(y_pad, dest, N):
    info = plsc.get_sparse_core_info()
    NC, NS = info.num_cores, info.num_subcores
    NW = NC * NS
    OPW = N // NW
    CH = 128
    mesh = plsc.VectorSubcoreMesh(core_axis_name="c", subcore_axis_name="s")

    @functools.partial(
        pl.kernel, mesh=mesh,
        out_type=jax.ShapeDtypeStruct((N, 128), jnp.float32),
        scratch_types=[
            pltpu.VMEM((CH,), jnp.int32),
            pltpu.VMEM((CH, 128), jnp.float32),
            pltpu.SemaphoreType.DMA,
        ],
    )
    def pick(y_hbm, dest_hbm, out_hbm, d_v, o_v, sem):
        wid = lax.axis_index("s") * NC + lax.axis_index("c")
        base = wid * OPW

        def body(i, carry):
            off = base + i * CH
            pltpu.sync_copy(dest_hbm.at[pl.ds(off, CH)], d_v)
            pltpu.async_copy(y_hbm.at[d_v], o_v, sem).wait()
            pltpu.sync_copy(o_v, out_hbm.at[pl.ds(off, CH)])
            return carry

        lax.fori_loop(0, OPW // CH, body, 0)

    return pick(y_pad, dest)


def _tc_fused(x_sorted, blk_e, W0, b0, g0, be0, W1, b1, g1, be1, W2, b2, NBLK):
    NPAD, D = x_sorted.shape
    E, H1, H2 = W1.shape
    inv = 1.0 / math.sqrt(1.0 + EPS)

    def body(se_ref, xs_ref, W0_ref, b0_ref, g0_ref, be0_ref,
             W1_ref, b1_ref, g1_ref, be1_ref, w2_ref, b2_ref, out_ref):
        h = jnp.dot(xs_ref[...], W0_ref[...], preferred_element_type=jnp.float32)
        h = jnp.maximum(h + b0_ref[...], 0.0)
        h = h * (inv * g0_ref[...]) + be0_ref[...]
        z = jnp.dot(h.astype(jnp.bfloat16), W1_ref[0],
                    preferred_element_type=jnp.float32)
        z = jnp.maximum(z + b1_ref[0], 0.0)
        z = z * (inv * g1_ref[0]) + be1_ref[0]
        y = jnp.sum(z * w2_ref[0], axis=1, keepdims=True) + b2_ref[0]
        out_ref[...] = jnp.broadcast_to(y, (B, 128))

    const = lambda b, se: (0, 0)
    exp2 = lambda b, se: (se[b], 0)
    exp3 = lambda b, se: (se[b], 0, 0)
    grid_spec = pltpu.PrefetchScalarGridSpec(
        num_scalar_prefetch=1,
        grid=(NBLK,),
        in_specs=[
            pl.BlockSpec((B, D), lambda b, se: (b, 0)),
            pl.BlockSpec((D, H1), const),
            pl.BlockSpec((1, H1), const),
            pl.BlockSpec((1, H1), const),
            pl.BlockSpec((1, H1), const),
            pl.BlockSpec((1, H1, H2), exp3),
            pl.BlockSpec((1, 1, H2), exp3),
            pl.BlockSpec((1, 1, H2), exp3),
            pl.BlockSpec((1, 1, H2), exp3),
            pl.BlockSpec((1, 1, H2), exp3),
            pl.BlockSpec((1, 1, 1), exp3),
        ],
        out_specs=pl.BlockSpec((B, 128), lambda b, se: (b, 0)),
    )
    return pl.pallas_call(
        body,
        grid_spec=grid_spec,
        out_shape=jax.ShapeDtypeStruct((NPAD, 128), jnp.float32),
        compiler_params=pltpu.CompilerParams(
            dimension_semantics=("arbitrary",)),
    )(blk_e, x_sorted, W0,
      b0.reshape(1, H1), g0.reshape(1, H1), be0.reshape(1, H1),
      W1, b1.reshape(E, 1, H2), g1.reshape(E, 1, H2), be1.reshape(E, 1, H2),
      W2[:, :, 0].reshape(E, 1, H2), b2.reshape(E, 1, 1))


def kernel(x, t, W0, b0, g0, be0, W1, b1, g1, be1, W2, b2):
    N, D = x.shape
    E, H1, H2 = W1.shape
    NBLK = N // B + E
    NPAD = NBLK * B
    t32 = t.astype(jnp.int32)

    dest, blk_e = _routing_tables(t32, N, E, NBLK)
    x_sorted = _sc_scatter_rows(x, dest, NPAD)
    y_pad = _tc_fused(x_sorted, blk_e, W0, b0, g0, be0,
                      W1.astype(jnp.bfloat16), b1, g1, be1, W2, b2, NBLK)
    return _sc_unpermute(y_pad, dest, N)[:, :1]


# speedup vs baseline: 1.0513x; 1.0513x over previous
import functools
import math

import jax
import jax.numpy as jnp
from jax import lax
from jax.experimental import pallas as pl
from jax.experimental.pallas import tpu as pltpu
from jax.experimental.pallas import tpu_sc as plsc

EPS = 1e-5
B = 256


def _routing_tables(t32, N, E, NBLK):
    oh = (t32[:, None] == jnp.arange(E, dtype=jnp.int32)[None, :]).astype(jnp.int32)
    csum = jnp.cumsum(oh, axis=0)
    counts = csum[-1]
    within = jnp.sum(oh * csum, axis=1) - 1
    padded = ((counts + B - 1) // B) * B
    po = jnp.concatenate([jnp.zeros((1,), jnp.int32),
                          jnp.cumsum(padded)[:-1].astype(jnp.int32)])
    dest = jnp.sum(oh * po[None, :], axis=1) + within
    blk_e = jnp.sum((jnp.arange(NBLK, dtype=jnp.int32)[None, :]
                     >= (po // B)[1:, None]).astype(jnp.int32), axis=0)
    return dest, blk_e


def _sc_scatter_rows(x, dest, NPAD):
    N, D = x.shape
    info = plsc.get_sparse_core_info()
    NC, NS = info.num_cores, info.num_subcores
    NW = NC * NS
    TPW = N // NW
    CH = 32
    NCH = TPW // CH
    mesh = plsc.VectorSubcoreMesh(core_axis_name="c", subcore_axis_name="s")
    dest3 = dest.reshape(NW, NCH, CH)

    @functools.partial(
        pl.kernel, mesh=mesh,
        out_type=jax.ShapeDtypeStruct((NPAD, D), x.dtype),
        scratch_types=[
            pltpu.VMEM((NCH, CH), jnp.int32),
            pltpu.VMEM((3, CH, D), x.dtype),
            pltpu.SemaphoreType.DMA,
            pltpu.SemaphoreType.DMA,
            pltpu.SemaphoreType.DMA,
            pltpu.SemaphoreType.DMA,
        ],
    )
    def scatter_rows(x_hbm, dest_hbm, out_hbm, d_v, rows_v,
                     sem_i0, sem_i1, sem_i2, sem_o):
        wid = lax.axis_index("s") * NC + lax.axis_index("c")
        base = wid * TPW
        sems = (sem_i0, sem_i1, sem_i2)

        def start_in(i, b):
            return pltpu.async_copy(
                x_hbm.at[pl.ds(base + i * CH, CH)], rows_v.at[b], sems[b])

        def start_out(i, b):
            return pltpu.async_copy(
                rows_v.at[b], out_hbm.at[d_v.at[i]], sem_o)

        NB = 3
        h_in = {0: start_in(0, 0), 1: start_in(1, 1)}
        pltpu.sync_copy(dest_hbm.at[wid], d_v)
        h_out = {}
        for i in range(NCH):
            b = i % NB
            h_in.pop(i).wait()
            if i + 2 < NCH:
                nb = (i + 2) % NB
                if h_out.get(nb) is not None:
                    h_out.pop(nb).wait()
                h_in[i + 2] = start_in(i + 2, nb)
            h_out[b] = start_out(i, b)
        for b, h in list(h_out.items()):
            h.wait()

    return scatter_rows(x, dest3)


def _sc_unpermute(y_pad, dest, N):
    info = plsc.get_sparse_core_info()
    NC, NS = info.num_cores, info.num_subcores
    NW = NC * NS
    OPW = N // NW
    CH = 128
    mesh = plsc.VectorSubcoreMesh(core_axis_name="c", subcore_axis_name="s")

    @functools.partial(
        pl.kernel, mesh=mesh,
        out_type=jax.ShapeDtypeStruct((N, 128), jnp.float32),
        scratch_types=[
            pltpu.VMEM((2, CH), jnp.int32),
            pltpu.VMEM((2, CH, 128), jnp.float32),
            pltpu.SemaphoreType.DMA,
            pltpu.SemaphoreType.DMA,
            pltpu.SemaphoreType.DMA,
        ],
    )
    def pick(y_hbm, dest_hbm, out_hbm, d_v, o_v, sem_d, sem_g, sem_w):
        wid = lax.axis_index("s") * NC + lax.axis_index("c")
        base = wid * OPW
        NCHU = OPW // CH
        hd = [pltpu.async_copy(dest_hbm.at[pl.ds(base + i * CH, CH)],
                               d_v.at[i], sem_d) for i in range(NCHU)]
        hg = [None] * NCHU
        hw = [None] * NCHU
        for i in range(NCHU):
            hd[i].wait()
            hg[i] = pltpu.async_copy(y_hbm.at[d_v.at[i]], o_v.at[i], sem_g)
        for i in range(NCHU):
            hg[i].wait()
            hw[i] = pltpu.async_copy(
                o_v.at[i], out_hbm.at[pl.ds(base + i * CH, CH)], sem_w)
        for i in range(NCHU):
            hw[i].wait()

    return pick(y_pad, dest)


def _tc_fused(x_sorted, blk_e, W0, b0, g0, be0, W1, b1, g1, be1, W2, b2, NBLK):
    NPAD, D = x_sorted.shape
    E, H1, H2 = W1.shape
    inv = 1.0 / math.sqrt(1.0 + EPS)

    def body(se_ref, xs_ref, W0_ref, b0_ref, g0_ref, be0_ref,
             W1_ref, b1_ref, g1_ref, be1_ref, w2_ref, b2_ref, out_ref):
        h = jnp.dot(xs_ref[...], W0_ref[...], preferred_element_type=jnp.float32)
        h = jnp.maximum(h + b0_ref[...], 0.0)
        h = h * (inv * g0_ref[...]) + be0_ref[...]
        z = jnp.dot(h.astype(jnp.bfloat16), W1_ref[0],
                    preferred_element_type=jnp.float32)
        z = jnp.maximum(z + b1_ref[0], 0.0)
        z = z * (inv * g1_ref[0]) + be1_ref[0]
        y = jnp.sum(z * w2_ref[0], axis=1, keepdims=True) + b2_ref[0]
        out_ref[...] = jnp.broadcast_to(y, (B, 128))

    const = lambda b, se: (0, 0)
    exp2 = lambda b, se: (se[b], 0)
    exp3 = lambda b, se: (se[b], 0, 0)
    grid_spec = pltpu.PrefetchScalarGridSpec(
        num_scalar_prefetch=1,
        grid=(NBLK,),
        in_specs=[
            pl.BlockSpec((B, D), lambda b, se: (b, 0)),
            pl.BlockSpec((D, H1), const),
            pl.BlockSpec((1, H1), const),
            pl.BlockSpec((1, H1), const),
            pl.BlockSpec((1, H1), const),
            pl.BlockSpec((1, H1, H2), exp3),
            pl.BlockSpec((1, 1, H2), exp3),
            pl.BlockSpec((1, 1, H2), exp3),
            pl.BlockSpec((1, 1, H2), exp3),
            pl.BlockSpec((1, 1, H2), exp3),
            pl.BlockSpec((1, 1, 1), exp3),
        ],
        out_specs=pl.BlockSpec((B, 128), lambda b, se: (b, 0)),
    )
    return pl.pallas_call(
        body,
        grid_spec=grid_spec,
        out_shape=jax.ShapeDtypeStruct((NPAD, 128), jnp.float32),
        compiler_params=pltpu.CompilerParams(
            dimension_semantics=("arbitrary",)),
    )(blk_e, x_sorted, W0,
      b0.reshape(1, H1), g0.reshape(1, H1), be0.reshape(1, H1),
      W1, b1.reshape(E, 1, H2), g1.reshape(E, 1, H2), be1.reshape(E, 1, H2),
      W2[:, :, 0].reshape(E, 1, H2), b2.reshape(E, 1, 1))


def kernel(x, t, W0, b0, g0, be0, W1, b1, g1, be1, W2, b2):
    N, D = x.shape
    E, H1, H2 = W1.shape
    NBLK = N // B + E
    NPAD = NBLK * B
    t32 = t.astype(jnp.int32)

    dest, blk_e = _routing_tables(t32, N, E, NBLK)
    x_sorted = _sc_scatter_rows(x, dest, NPAD)
    y_pad = _tc_fused(x_sorted, blk_e, W0, b0, g0, be0,
                      W1, b1, g1, be1, W2, b2, NBLK)
    return _sc_unpermute(y_pad, dest, N)[:, :1]


# speedup vs baseline: 1.0548x; 1.0034x over previous
import functools
import math

import jax
import jax.numpy as jnp
from jax import lax
from jax.experimental import pallas as pl
from jax.experimental.pallas import tpu as pltpu
from jax.experimental.pallas import tpu_sc as plsc

EPS = 1e-5
B = 256


def _routing_tables(t32, N, E, NBLK):
    oh = (t32[:, None] == jnp.arange(E, dtype=jnp.int32)[None, :]).astype(jnp.int32)
    csum = jnp.cumsum(oh, axis=0)
    counts = csum[-1]
    within = jnp.sum(oh * csum, axis=1) - 1
    padded = ((counts + B - 1) // B) * B
    po = jnp.concatenate([jnp.zeros((1,), jnp.int32),
                          jnp.cumsum(padded)[:-1].astype(jnp.int32)])
    dest = jnp.sum(oh * po[None, :], axis=1) + within
    blk_e = jnp.sum((jnp.arange(NBLK, dtype=jnp.int32)[None, :]
                     >= (po // B)[1:, None]).astype(jnp.int32), axis=0)
    return dest, blk_e


def _sc_scatter_rows(x, dest, NPAD):
    N, D = x.shape
    info = plsc.get_sparse_core_info()
    NC, NS = info.num_cores, info.num_subcores
    NW = NC * NS
    TPW = N // NW
    CH = 32
    NCH = TPW // CH
    mesh = plsc.VectorSubcoreMesh(core_axis_name="c", subcore_axis_name="s")
    dest3 = dest.reshape(NW, NCH, CH)

    @functools.partial(
        pl.kernel, mesh=mesh,
        out_type=jax.ShapeDtypeStruct((NPAD, D), x.dtype),
        scratch_types=[
            pltpu.VMEM((NCH, CH), jnp.int32),
            pltpu.VMEM((3, CH, D), x.dtype),
            pltpu.SemaphoreType.DMA,
            pltpu.SemaphoreType.DMA,
            pltpu.SemaphoreType.DMA,
            pltpu.SemaphoreType.DMA,
        ],
    )
    def scatter_rows(x_hbm, dest_hbm, out_hbm, d_v, rows_v,
                     sem_i0, sem_i1, sem_i2, sem_o):
        wid = lax.axis_index("s") * NC + lax.axis_index("c")
        base = wid * TPW
        sems = (sem_i0, sem_i1, sem_i2)

        def start_in(i, b):
            return pltpu.async_copy(
                x_hbm.at[pl.ds(base + i * CH, CH)], rows_v.at[b], sems[b])

        def start_out(i, b):
            return pltpu.async_copy(
                rows_v.at[b], out_hbm.at[d_v.at[i]], sem_o)

        NB = 3
        h_in = {0: start_in(0, 0), 1: start_in(1, 1)}
        pltpu.sync_copy(dest_hbm.at[wid], d_v)
        h_out = {}
        for i in range(NCH):
            b = i % NB
            h_in.pop(i).wait()
            if i + 2 < NCH:
                nb = (i + 2) % NB
                if h_out.get(nb) is not None:
                    h_out.pop(nb).wait()
                h_in[i + 2] = start_in(i + 2, nb)
            h_out[b] = start_out(i, b)
        for b, h in list(h_out.items()):
            h.wait()

    return scatter_rows(x, dest3)


def _sc_unpermute(y_pad, dest, N):
    info = plsc.get_sparse_core_info()
    NC, NS = info.num_cores, info.num_subcores
    NW = NC * NS
    OPW = N // NW
    CH = 128
    mesh = plsc.VectorSubcoreMesh(core_axis_name="c", subcore_axis_name="s")

    @functools.partial(
        pl.kernel, mesh=mesh,
        out_type=jax.ShapeDtypeStruct((N, 128), jnp.float32),
        scratch_types=[
            pltpu.VMEM((2, CH), jnp.int32),
            pltpu.VMEM((2, CH, 128), jnp.float32),
            pltpu.SemaphoreType.DMA,
            pltpu.SemaphoreType.DMA,
            pltpu.SemaphoreType.DMA,
        ],
    )
    def pick(y_hbm, dest_hbm, out_hbm, d_v, o_v, sem_d, sem_g, sem_w):
        wid = lax.axis_index("s") * NC + lax.axis_index("c")
        base = wid * OPW
        NCHU = OPW // CH
        hd = [pltpu.async_copy(dest_hbm.at[pl.ds(base + i * CH, CH)],
                               d_v.at[i], sem_d) for i in range(NCHU)]
        hg = [None] * NCHU
        hw = [None] * NCHU
        for i in range(NCHU):
            hd[i].wait()
            hg[i] = pltpu.async_copy(y_hbm.at[d_v.at[i]], o_v.at[i], sem_g)
        for i in range(NCHU):
            hg[i].wait()
            hw[i] = pltpu.async_copy(
                o_v.at[i], out_hbm.at[pl.ds(base + i * CH, CH)], sem_w)
        for i in range(NCHU):
            hw[i].wait()

    return pick(y_pad, dest)


def _tc_fused(x_sorted, blk_e, W0, b0, g0, be0, W1, b1, g1, be1, W2, b2, NBLK):
    NPAD, D = x_sorted.shape
    E, H1, H2 = W1.shape
    inv = 1.0 / math.sqrt(1.0 + EPS)

    def body(se_ref, xs_ref, W0_ref, a0_ref, W1_ref, a1_ref, out_ref):
        a0 = a0_ref[...]
        h = jnp.dot(xs_ref[...], W0_ref[...], preferred_element_type=jnp.float32)
        h = jnp.maximum(h + a0[0:1], 0.0)
        h = h * (inv * a0[1:2]) + a0[2:3]
        z = jnp.dot(h.astype(jnp.bfloat16), W1_ref[0],
                    preferred_element_type=jnp.float32)
        a1 = a1_ref[0]
        z = jnp.maximum(z + a1[0:1], 0.0)
        z = z * (inv * a1[1:2]) + a1[2:3]
        y = jnp.sum(z * a1[3:4], axis=1, keepdims=True) + a1[4:5, 0:1]
        out_ref[...] = jnp.broadcast_to(y, (B, 128))

    const = lambda b, se: (0, 0)
    exp3 = lambda b, se: (se[b], 0, 0)
    grid_spec = pltpu.PrefetchScalarGridSpec(
        num_scalar_prefetch=1,
        grid=(NBLK,),
        in_specs=[
            pl.BlockSpec((B, D), lambda b, se: (b, 0)),
            pl.BlockSpec((D, H1), const),
            pl.BlockSpec((3, H1), const),
            pl.BlockSpec((1, H1, H2), exp3),
            pl.BlockSpec((1, 5, H2), exp3),
        ],
        out_specs=pl.BlockSpec((B, 128), lambda b, se: (b, 0)),
    )
    aux0 = jnp.stack([b0, g0, be0], axis=0)
    aux1 = jnp.stack([b1, g1, be1, W2[:, :, 0],
                      jnp.broadcast_to(b2, (E, H2))], axis=1)
    return pl.pallas_call(
        body,
        grid_spec=grid_spec,
        out_shape=jax.ShapeDtypeStruct((NPAD, 128), jnp.float32),
        compiler_params=pltpu.CompilerParams(
            dimension_semantics=("arbitrary",)),
    )(blk_e, x_sorted, W0, aux0, W1, aux1)


def kernel(x, t, W0, b0, g0, be0, W1, b1, g1, be1, W2, b2):
    N, D = x.shape
    E, H1, H2 = W1.shape
    NBLK = N // B + E
    NPAD = NBLK * B
    t32 = t.astype(jnp.int32)

    dest, blk_e = _routing_tables(t32, N, E, NBLK)
    x_sorted = _sc_scatter_rows(x, dest, NPAD)
    y_pad = _tc_fused(x_sorted, blk_e, W0, b0, g0, be0,
                      W1, b1, g1, be1, W2, b2, NBLK)
    return _sc_unpermute(y_pad, dest, N)[:, :1]
